# Initial kernel scaffold; baseline (speedup 1.0000x reference)
import jax
import jax.numpy as jnp
from jax.experimental import pallas as pl


def kernel(x, x_p, edge_index):
    raise NotImplementedError("write your pallas kernel here")


# speedup vs baseline: 5.6319x; 5.6319x over previous
import functools

import jax
import jax.numpy as jnp
from jax import lax
from jax.experimental import pallas as pl
from jax.experimental.pallas import tpu as pltpu, tpu_sc as plsc

C = 128
K = 16
LANES = 16
CH = C // LANES


@functools.cache
def _make_sc_kernel(n_pad: int, npt: int):
    info = plsc.get_sparse_core_info()
    nc = info.num_cores
    mesh = plsc.VectorSubcoreMesh(core_axis_name="c", subcore_axis_name="s")
    inv_c = 1.0 / C
    c1 = 1e-6
    c2 = 1e-6

    take_dnums = lax.GatherDimensionNumbers(
        offset_dims=(), collapsed_slice_dims=(0,), start_index_map=(0,)
    )

    def lane_take(v, perm):
        return lax.gather(
            v, perm[:, None], take_dnums, slice_sizes=(1,),
            mode=lax.GatherScatterMode.PROMISE_IN_BOUNDS,
        )

    def allsum(v):
        lane = lax.iota(jnp.int32, LANES)
        for sh in (8, 4, 2, 1):
            v = v + lane_take(v, lax.bitwise_xor(lane, sh))
        return v

    @functools.partial(
        pl.kernel,
        out_type=jax.ShapeDtypeStruct((n_pad, C), jnp.float32),
        mesh=mesh,
        scratch_types=[
            pltpu.VMEM((npt, 2 * K), jnp.int32),
            pltpu.VMEM((2 * K, 2 * C), jnp.float32),
            pltpu.VMEM((npt, C), jnp.float32),
            pltpu.SemaphoreType.DMA,
        ],
    )
    def sc_kernel(tbl_hbm, eidx_hbm, out_hbm, eidx_v, rows_v, out_v, sem):
        wid = lax.axis_index("s") * nc + lax.axis_index("c")
        base = wid * npt
        pltpu.sync_copy(eidx_hbm.at[pl.ds(base, npt)], eidx_v)

        def node_body(nn, carry):
            pltpu.async_copy(tbl_hbm.at[eidx_v.at[nn]], rows_v, sem).wait()

            def edge_body(kk, accs):
                a = rows_v[kk, pl.ds(0, LANES)]
                b = rows_v[kk + K, pl.ds(0, LANES)]
                dotv = a * b
                siv = a
                ssiv = a * a
                sjv = b
                ssjv = b * b
                for cc in range(1, CH):
                    a = rows_v[kk, pl.ds(cc * LANES, LANES)]
                    b = rows_v[kk + K, pl.ds(cc * LANES, LANES)]
                    dotv = dotv + a * b
                    siv = siv + a
                    ssiv = ssiv + a * a
                    sjv = sjv + b
                    ssjv = ssjv + b * b
                mi = allsum(siv) * inv_c
                mj = allsum(sjv) * inv_c
                vi = allsum(ssiv) * inv_c - mi * mi
                vj = allsum(ssjv) * inv_c - mj * mj
                cov = allsum(dotv) * inv_c - mi * mj
                s1 = (2.0 * mi * mj + c1) / (mi * mi + mj * mj + c1)
                s2 = (2.0 * cov + c2) / (vi + vj + c2)
                sff = 1.0 - s1 * s2
                out = []
                for cc in range(CH):
                    pi = rows_v[kk, pl.ds(C + cc * LANES, LANES)]
                    pj = rows_v[kk + K, pl.ds(C + cc * LANES, LANES)]
                    out.append(accs[cc] + (pi + pj + jnp.abs(pi - pj) * sff))
                return tuple(out)

            zeros = tuple(jnp.zeros((LANES,), jnp.float32) for _ in range(CH))
            accs = lax.fori_loop(0, K, edge_body, zeros)
            for cc in range(CH):
                out_v[nn, pl.ds(cc * LANES, LANES)] = accs[cc]
            return carry

        lax.fori_loop(0, npt, node_body, 0)
        pltpu.sync_copy(out_v, out_hbm.at[pl.ds(base, npt)])

    return sc_kernel


def kernel(x, x_p, edge_index):
    n = x.shape[2]
    x2 = x[0, :, :, 0]
    xp2 = x_p[0, :, :, 0]
    tbl = jnp.concatenate([x2.T, xp2.T], axis=1)
    e = edge_index[:, 0].astype(jnp.int32)
    eidx = jnp.concatenate([e[1], e[0]], axis=1)

    info = plsc.get_sparse_core_info()
    nw = info.num_cores * info.num_subcores
    npt = -(-n // (nw * 8)) * 8
    n_pad = npt * nw
    eidx = jnp.pad(eidx, ((0, n_pad - n), (0, 0)))

    out_t = _make_sc_kernel(n_pad, npt)(tbl, eidx)
    return out_t[:n].T[None, :, :, None]


# speedup vs baseline: 8.0888x; 1.4362x over previous
import functools

import jax
import jax.numpy as jnp
from jax import lax
from jax.experimental import pallas as pl
from jax.experimental.pallas import tpu as pltpu, tpu_sc as plsc

C = 128
K = 16
LANES = 16
CH = C // LANES


@functools.cache
def _make_sc_kernel(n_pad: int, npt: int):
    info = plsc.get_sparse_core_info()
    nc = info.num_cores
    mesh = plsc.VectorSubcoreMesh(core_axis_name="c", subcore_axis_name="s")
    inv_c = 1.0 / C
    c1 = 1e-6
    c2 = 1e-6

    take_dnums = lax.GatherDimensionNumbers(
        offset_dims=(), collapsed_slice_dims=(0,), start_index_map=(0,)
    )

    def lane_take(v, perm):
        return lax.gather(
            v, perm[:, None], take_dnums, slice_sizes=(1,),
            mode=lax.GatherScatterMode.PROMISE_IN_BOUNDS,
        )

    def allsum(v):
        lane = lax.iota(jnp.int32, LANES)
        for sh in (8, 4, 2, 1):
            v = v + lane_take(v, lax.bitwise_xor(lane, sh))
        return v

    DEPTH = 4

    @functools.partial(
        pl.kernel,
        out_type=jax.ShapeDtypeStruct((n_pad, C), jnp.float32),
        mesh=mesh,
        scratch_types=[
            pltpu.VMEM((npt, 2 * K), jnp.int32),
            pltpu.VMEM((npt, C), jnp.float32),
        ]
        + [pltpu.VMEM((2 * K, 2 * C), jnp.float32) for _ in range(DEPTH)]
        + [pltpu.SemaphoreType.DMA for _ in range(DEPTH)],
    )
    def sc_kernel(tbl_hbm, eidx_hbm, out_hbm, eidx_v, out_v, *ring):
        rows = ring[:DEPTH]
        sems = ring[DEPTH:]
        wid = lax.axis_index("s") * nc + lax.axis_index("c")
        base = wid * npt
        pltpu.sync_copy(eidx_hbm.at[pl.ds(base, npt)], eidx_v)

        def issue(nn, b):
            pltpu.async_copy(tbl_hbm.at[eidx_v.at[nn]], rows[b], sems[b])

        def slot_wait(b):
            pltpu.make_async_copy(tbl_hbm.at[pl.ds(0, 2 * K)], rows[b], sems[b]).wait()

        def compute(nn, rows_v):
            def edge_body(kk, accs):
                a = rows_v[kk, pl.ds(0, LANES)]
                b = rows_v[kk + K, pl.ds(0, LANES)]
                dotv = a * b
                siv = a
                ssiv = a * a
                sjv = b
                ssjv = b * b
                for cc in range(1, CH):
                    a = rows_v[kk, pl.ds(cc * LANES, LANES)]
                    b = rows_v[kk + K, pl.ds(cc * LANES, LANES)]
                    dotv = dotv + a * b
                    siv = siv + a
                    ssiv = ssiv + a * a
                    sjv = sjv + b
                    ssjv = ssjv + b * b
                mi = allsum(siv) * inv_c
                mj = allsum(sjv) * inv_c
                vi = allsum(ssiv) * inv_c - mi * mi
                vj = allsum(ssjv) * inv_c - mj * mj
                cov = allsum(dotv) * inv_c - mi * mj
                s1 = (2.0 * mi * mj + c1) / (mi * mi + mj * mj + c1)
                s2 = (2.0 * cov + c2) / (vi + vj + c2)
                sff = 1.0 - s1 * s2
                out = []
                for cc in range(CH):
                    pi = rows_v[kk, pl.ds(C + cc * LANES, LANES)]
                    pj = rows_v[kk + K, pl.ds(C + cc * LANES, LANES)]
                    out.append(accs[cc] + (pi + pj + jnp.abs(pi - pj) * sff))
                return tuple(out)

            zeros = tuple(jnp.zeros((LANES,), jnp.float32) for _ in range(CH))
            accs = lax.fori_loop(0, K, edge_body, zeros)
            for cc in range(CH):
                out_v[nn, pl.ds(cc * LANES, LANES)] = accs[cc]

        for b in range(DEPTH):
            issue(b, b)

        def outer(g, carry):
            for b in range(DEPTH):
                nn = g * DEPTH + b
                slot_wait(b)
                compute(nn, rows[b])

                @pl.when(nn + DEPTH < npt)
                def _():
                    issue(nn + DEPTH, b)

            return carry

        lax.fori_loop(0, npt // DEPTH, outer, 0)
        pltpu.sync_copy(out_v, out_hbm.at[pl.ds(base, npt)])

    return sc_kernel


def kernel(x, x_p, edge_index):
    n = x.shape[2]
    x2 = x[0, :, :, 0]
    xp2 = x_p[0, :, :, 0]
    tbl = jnp.concatenate([x2.T, xp2.T], axis=1)
    e = edge_index[:, 0].astype(jnp.int32)
    eidx = jnp.concatenate([e[1], e[0]], axis=1)

    info = plsc.get_sparse_core_info()
    nw = info.num_cores * info.num_subcores
    npt = -(-n // (nw * 8)) * 8
    n_pad = npt * nw
    eidx = jnp.pad(eidx, ((0, n_pad - n), (0, 0)))

    out_t = _make_sc_kernel(n_pad, npt)(tbl, eidx)
    return out_t[:n].T[None, :, :, None]


# speedup vs baseline: 8.3294x; 1.0297x over previous
import functools

import jax
import jax.numpy as jnp
from jax import lax
from jax.experimental import pallas as pl
from jax.experimental.pallas import tpu as pltpu, tpu_sc as plsc

C = 128
K = 16
LANES = 16
CH = C // LANES
W = 2 * C


@functools.cache
def _make_tc_stats(n_pad: int):
    inv_c = 1.0 / C

    def body(x_ref, o_ref):
        xb = x_ref[...]
        m = jnp.sum(xb, axis=0) * inv_c
        v = jnp.sum(xb * xb, axis=0) * inv_c - m * m
        o_ref[0, :] = m
        o_ref[1, :] = v

    return pl.pallas_call(
        body, out_shape=jax.ShapeDtypeStruct((8, n_pad), jnp.float32)
    )


@functools.cache
def _make_sc_kernel(n_pad: int, npt: int):
    info = plsc.get_sparse_core_info()
    nc = info.num_cores
    mesh = plsc.VectorSubcoreMesh(core_axis_name="c", subcore_axis_name="s")
    inv_c = 1.0 / C
    c1 = 1e-6
    c2 = 1e-6

    take_dnums = lax.GatherDimensionNumbers(
        offset_dims=(), collapsed_slice_dims=(0,), start_index_map=(0,)
    )

    def lane_take(v, perm):
        return lax.gather(
            v, perm[:, None], take_dnums, slice_sizes=(1,),
            mode=lax.GatherScatterMode.PROMISE_IN_BOUNDS,
        )

    def allsum(v):
        lane = lax.iota(jnp.int32, LANES)
        for sh in (8, 4, 2, 1):
            v = v + lane_take(v, lax.bitwise_xor(lane, sh))
        return v

    DEPTH = 4

    @functools.partial(
        pl.kernel,
        out_type=jax.ShapeDtypeStruct((n_pad, C), jnp.float32),
        mesh=mesh,
        scratch_types=[
            pltpu.VMEM((npt * 2 * K,), jnp.int32),
            pltpu.VMEM((npt, C), jnp.float32),
            pltpu.VMEM((n_pad,), jnp.float32),
            pltpu.VMEM((n_pad,), jnp.float32),
        ]
        + [pltpu.VMEM((2 * K, W), jnp.float32) for _ in range(DEPTH)]
        + [pltpu.SemaphoreType.DMA for _ in range(DEPTH)],
        compiler_params=pltpu.CompilerParams(needs_layout_passes=False),
    )
    def sc_kernel(tbl_hbm, eidx_hbm, m_hbm, v_hbm, out_hbm, eidx_v, out_v, m_v, v_v, *ring):
        rows = ring[:DEPTH]
        sems = ring[DEPTH:]
        wid = lax.axis_index("s") * nc + lax.axis_index("c")
        base = wid * npt
        pltpu.sync_copy(eidx_hbm.at[pl.ds(base * 2 * K, npt * 2 * K)], eidx_v)
        pltpu.sync_copy(m_hbm, m_v)
        pltpu.sync_copy(v_hbm, v_v)

        def issue(nn, b):
            pltpu.async_copy(tbl_hbm.at[eidx_v.at[pl.ds(nn * 2 * K, 2 * K)]], rows[b], sems[b])

        def slot_wait(b):
            pltpu.make_async_copy(tbl_hbm.at[pl.ds(0, 2 * K)], rows[b], sems[b]).wait()

        def compute(nn, rows_v):
            idx_i = eidx_v[pl.ds(nn * 2 * K, LANES)]
            idx_j = eidx_v[pl.ds(nn * 2 * K + K, LANES)]
            mi = plsc.load_gather(m_v, [idx_i])
            vi = plsc.load_gather(v_v, [idx_i])
            mj = plsc.load_gather(m_v, [idx_j])
            vj = plsc.load_gather(v_v, [idx_j])
            mimj = mi * mj
            num1 = 2.0 * mimj + c1
            den1 = mi * mi + mj * mj + c1
            den2 = vi + vj + c2
            r12 = num1 / (den1 * den2)

            def edge_body(kk, accs):
                bidx = jnp.full((LANES,), kk, jnp.int32)
                dotv = None
                dv = []
                out = []
                for cc in range(CH):
                    a = rows_v[kk, pl.ds(cc * LANES, LANES)]
                    b = rows_v[kk + K, pl.ds(cc * LANES, LANES)]
                    dotv = a * b if dotv is None else dotv + a * b
                    pi = rows_v[kk, pl.ds(C + cc * LANES, LANES)]
                    pj = rows_v[kk + K, pl.ds(C + cc * LANES, LANES)]
                    out.append(accs[cc] + (pi + pj))
                    dv.append(jnp.abs(pi - pj))
                mimj_b = lane_take(mimj, bidx)
                r12_b = lane_take(r12, bidx)
                cov2 = 2.0 * (allsum(dotv) * inv_c - mimj_b) + c2
                sff = 1.0 - r12_b * cov2
                return tuple(out[cc] + dv[cc] * sff for cc in range(CH))

            zeros = tuple(jnp.zeros((LANES,), jnp.float32) for _ in range(CH))
            accs = lax.fori_loop(0, K, edge_body, zeros)
            for cc in range(CH):
                out_v[nn, pl.ds(cc * LANES, LANES)] = accs[cc]

        for b in range(DEPTH):
            issue(b, b)

        def outer(g, carry):
            for b in range(DEPTH):
                nn = g * DEPTH + b
                slot_wait(b)
                compute(nn, rows[b])

                @pl.when(nn + DEPTH < npt)
                def _():
                    issue(nn + DEPTH, b)

            return carry

        lax.fori_loop(0, npt // DEPTH, outer, 0)
        pltpu.sync_copy(out_v, out_hbm.at[pl.ds(base, npt)])

    return sc_kernel


def kernel(x, x_p, edge_index):
    n = x.shape[2]
    x2 = x[0, :, :, 0]
    xp2 = x_p[0, :, :, 0]
    e = edge_index[:, 0].astype(jnp.int32)
    eidx = jnp.concatenate([e[1], e[0]], axis=1)

    info = plsc.get_sparse_core_info()
    nw = info.num_cores * info.num_subcores
    npt = -(-n // (nw * 8)) * 8
    n_pad = npt * nw
    eidx = jnp.pad(eidx, ((0, n_pad - n), (0, 0))).reshape(-1)

    x2p = jnp.pad(x2, ((0, 0), (0, n_pad - n)))
    mv = _make_tc_stats(n_pad)(x2p)
    tbl = jnp.concatenate([x2.T, xp2.T], axis=1)
    out_t = _make_sc_kernel(n_pad, npt)(tbl, eidx, mv[0], mv[1])
    return out_t[:n].T[None, :, :, None]
